# baseline (speedup 1.0000x reference)
import functools

import jax
import jax.numpy as jnp
from jax import lax
from jax.experimental import pallas as pl
from jax.experimental.pallas import tpu as pltpu
from jax.experimental.pallas import tpu_sc as plsc

N = 10000
F = 128
DIM = 32
G = 64
C = 10
E = 320000
L = 3

NPAD = 10112
TRASH = N
NC = 2
NS = 16
MACROS = 8
GCH = 10
GPT = MACROS * GCH
E_PAD = NC * NS * GPT * 128
EG = E_PAD // 128
RPT = NPAD // NS
RP = NPAD // 4

_mesh = plsc.VectorSubcoreMesh(core_axis_name="c", subcore_axis_name="s", num_cores=NC)


@functools.partial(
    pl.kernel,
    out_type=jax.ShapeDtypeStruct((NC, NPAD, DIM), jnp.float32),
    mesh=_mesh,
    scratch_types=[
        pltpu.VMEM((3, GCH, 128), jnp.int32),
        pltpu.VMEM((3, GCH, 128), jnp.int32),
        pltpu.VMEM((2, GCH, 128, DIM), jnp.float32),
        pltpu.VMEM_SHARED((NPAD, DIM), jnp.float32),
        pltpu.VMEM_SHARED((NPAD, DIM), jnp.float32),
        pltpu.SemaphoreType.DMA,
        pltpu.SemaphoreType.DMA,
        pltpu.SemaphoreType.DMA,
    ],
    compiler_params=pltpu.CompilerParams(use_tc_tiling_on_sc=False),
)
def _sc_agg(row_hbm, col_hbm, hw_hbm, out_hbm, colbuf, rowbuf, vals, acc,
            hw_s, sem_i, sem_g, sem_s):
    cid = lax.axis_index("c")
    sid = lax.axis_index("s")
    wid = sid * NC + cid
    pltpu.sync_copy(hw_hbm.at[pl.ds(sid * RPT, RPT)],
                    acc.at[pl.ds(sid * RPT, RPT)])
    pltpu.sync_copy(hw_hbm.at[pl.ds(sid * RPT, RPT)],
                    hw_s.at[pl.ds(sid * RPT, RPT)])

    def idx_copies(m):
        q = m % 3
        gbase = wid * GPT + m * GCH
        return (
            pltpu.make_async_copy(col_hbm.at[pl.ds(gbase, GCH)],
                                  colbuf.at[q], sem_i),
            pltpu.make_async_copy(row_hbm.at[pl.ds(gbase, GCH)],
                                  rowbuf.at[q], sem_i),
        )

    def gather_copies(m):
        q, b = m % 3, m % 2
        return [pltpu.make_async_copy(hw_s.at[colbuf.at[q, g]],
                                      vals.at[b, g], sem_g)
                for g in range(GCH)]

    def scatter_copies(m):
        q, b = m % 3, m % 2
        return [pltpu.make_async_copy(vals.at[b, g],
                                      acc.at[rowbuf.at[q, g]], sem_s)
                for g in range(GCH)]

    for c in idx_copies(0):
        c.start()
    plsc.subcore_barrier()
    for m in range(MACROS):
        if m >= 2:
            for c in scatter_copies(m - 2):
                c.wait()
        for c in idx_copies(m):
            c.wait()
        if m + 1 < MACROS:
            for c in idx_copies(m + 1):
                c.start()
        for c in gather_copies(m):
            c.start()
        if m >= 1:
            for c in gather_copies(m - 1):
                c.wait()
            for c in scatter_copies(m - 1):
                c.start(add=True)
    for c in gather_copies(MACROS - 1):
        c.wait()
    for c in scatter_copies(MACROS - 1):
        c.start(add=True)
    for c in scatter_copies(MACROS - 2):
        c.wait()
    for c in scatter_copies(MACROS - 1):
        c.wait()
    plsc.subcore_barrier()
    pltpu.sync_copy(acc.at[pl.ds(sid * RPT, RPT)],
                    out_hbm.at[cid, pl.ds(sid * RPT, RPT)])


def _slot_pool(b4, h_p, width):
    seg = lax.broadcasted_iota(jnp.int32, (1, G), 1)
    s = jnp.zeros((G, width), jnp.float32)
    cnt = jnp.zeros((G, 1), jnp.float32)
    for j in range(4):
        Pj = (b4[:, j:j + 1] == seg).astype(jnp.float32)
        s = s + lax.dot_general(Pj, h_p[:, j * width:(j + 1) * width],
                                (((0,), (0,)), ((), ())),
                                preferred_element_type=jnp.float32)
        cnt = cnt + jnp.sum(Pj, axis=0)[:, None]
    return s / jnp.maximum(cnt, 1.0)


def _blockdiag(w, n_in, n_out):
    z = jnp.zeros((n_in, n_out), jnp.float32)
    rows = [jnp.concatenate([w if i == j else z for j in range(4)], axis=1)
            for i in range(4)]
    return jnp.concatenate(rows, axis=0)


def _head_body(x4_ref, w0_ref, b4_ref, hw_ref, pool_ref):
    x4 = x4_ref[...]
    hw_ref[...] = jnp.dot(x4, _blockdiag(w0_ref[...], F, DIM),
                          preferred_element_type=jnp.float32)
    pool_ref[...] = _slot_pool(b4_ref[...], x4, F)


_head = pl.pallas_call(
    _head_body,
    out_shape=(jax.ShapeDtypeStruct((RP, 4 * DIM), jnp.float32),
               jax.ShapeDtypeStruct((G, F), jnp.float32)),
)


def _bn_relu_packed(t, gamma, beta, mix):
    riota = lax.broadcasted_iota(jnp.int32, (RP, 1), 0)
    u = jnp.where(riota < N // 4, jnp.maximum(t, 0.0), 0.0)
    s1 = jnp.sum(u, axis=0, keepdims=True)
    s2 = jnp.sum(u * u, axis=0, keepdims=True)
    mu = jnp.dot(s1, mix, preferred_element_type=jnp.float32) / N
    m2 = jnp.dot(s2, mix, preferred_element_type=jnp.float32) / N
    var = m2 - mu * mu
    return gamma * (u - mu) / jnp.sqrt(var + 1e-5) + beta


def _tile4(v):
    return jnp.concatenate([v, v, v, v], axis=1)


def _mlp_body(parts_ref, hw_ref, b0_ref, g0_ref, be0_ref,
              w1_ref, b1_ref, g1_ref, be1_ref, wn_ref, b4_ref,
              hwn_ref, pool_ref):
    i = lax.broadcasted_iota(jnp.int32, (DIM, DIM), 0)
    j = lax.broadcasted_iota(jnp.int32, (DIM, DIM), 1)
    eye = (i == j).astype(jnp.float32)
    mix = jnp.concatenate([jnp.concatenate([eye] * 4, axis=1)] * 4, axis=0)
    agg = parts_ref[0] + parts_ref[1] - hw_ref[...]
    h = _bn_relu_packed(agg + _tile4(b0_ref[...]), _tile4(g0_ref[...]),
                        _tile4(be0_ref[...]), mix)
    t = jnp.dot(h, _blockdiag(w1_ref[...], DIM, DIM),
                preferred_element_type=jnp.float32) + _tile4(b1_ref[...])
    h2 = _bn_relu_packed(t, _tile4(g1_ref[...]), _tile4(be1_ref[...]), mix)
    hwn_ref[...] = jnp.dot(h2, _blockdiag(wn_ref[...], DIM, DIM),
                           preferred_element_type=jnp.float32)
    pool_ref[...] = _slot_pool(b4_ref[...], h2, DIM)


_mlp = pl.pallas_call(
    _mlp_body,
    out_shape=(jax.ShapeDtypeStruct((RP, 4 * DIM), jnp.float32),
               jax.ShapeDtypeStruct((G, DIM), jnp.float32)),
)


def _cls_body(px_ref, q0_ref, q1_ref, q2_ref, wc1_ref, bc1_ref,
              wc2_ref, bc2_ref, out_ref):
    z = jnp.concatenate(
        [px_ref[...], q0_ref[...], q1_ref[...], q2_ref[...]], axis=1)
    h = jnp.maximum(
        jnp.dot(z, wc1_ref[...], preferred_element_type=jnp.float32)
        + bc1_ref[...], 0.0)
    o = (jnp.dot(h, wc2_ref[...], preferred_element_type=jnp.float32)
         + bc2_ref[...])
    m = jnp.max(o, axis=1, keepdims=True)
    e = jnp.exp(o - m)
    out_ref[...] = (o - m) - jnp.log(jnp.sum(e, axis=1, keepdims=True))


_cls = pl.pallas_call(
    _cls_body,
    out_shape=jax.ShapeDtypeStruct((G, C), jnp.float32),
)


def kernel(x, edge_index, batch, params):
    row = edge_index[0]
    col = edge_index[1]
    pad = E_PAD - E
    rowadj = jnp.where(row == col, TRASH, row)
    rowadj2d = jnp.concatenate(
        [rowadj, jnp.full((pad,), TRASH, jnp.int32)]).reshape(EG, 128)
    col2d = jnp.concatenate(
        [col, jnp.zeros((pad,), jnp.int32)]).reshape(EG, 128)
    p = params

    x4 = jnp.concatenate(
        [x, jnp.zeros((NPAD - N, F), jnp.float32)]).reshape(RP, 4 * F)
    b4 = jnp.concatenate(
        [batch, jnp.full((NPAD - N,), G, jnp.int32)]).reshape(RP, 4)
    hw, poolx = _head(x4, p["W0_0"], b4)
    pools = [poolx]
    for k in range(L):
        parts = _sc_agg(rowadj2d, col2d, hw.reshape(NPAD, DIM))
        wn = p[f"W{k + 1}_0"] if k < L - 1 else p["W1_0"]
        hw, pk = _mlp(
            parts.reshape(NC, RP, 4 * DIM), hw,
            p[f"b{k}_0"][None], p[f"g{k}_0"][None], p[f"be{k}_0"][None],
            p[f"W{k}_1"],
            p[f"b{k}_1"][None], p[f"g{k}_1"][None], p[f"be{k}_1"][None],
            wn, b4)
        pools.append(pk)
    return _cls(pools[0], pools[1], pools[2], pools[3],
                p["Wc1"], p["bc1"][None], p["Wc2"], p["bc2"][None])


# speedup vs baseline: 1.0173x; 1.0173x over previous
import functools

import jax
import jax.numpy as jnp
from jax import lax
from jax.experimental import pallas as pl
from jax.experimental.pallas import tpu as pltpu
from jax.experimental.pallas import tpu_sc as plsc

N = 10000
F = 128
DIM = 32
G = 64
C = 10
E = 320000
L = 3

NPAD = 10112
TRASH = N
NC = 2
NS = 16
MACROS = 16
GCH = 5
GPT = MACROS * GCH
EG = E // 128
RPT = NPAD // NS
RP = NPAD // 4

_mesh = plsc.VectorSubcoreMesh(core_axis_name="c", subcore_axis_name="s", num_cores=NC)


@functools.partial(
    pl.kernel,
    out_type=jax.ShapeDtypeStruct((NC, NPAD, DIM), jnp.float32),
    mesh=_mesh,
    scratch_types=[
        pltpu.VMEM((3, GCH, 128), jnp.int32),
        pltpu.VMEM((3, GCH, 128), jnp.int32),
        pltpu.VMEM((2, GCH, 128, DIM), jnp.float32),
        pltpu.VMEM_SHARED((NPAD, DIM), jnp.float32),
        pltpu.VMEM_SHARED((NPAD, DIM), jnp.float32),
        pltpu.SemaphoreType.DMA,
        pltpu.SemaphoreType.DMA,
        pltpu.SemaphoreType.DMA,
    ],
    compiler_params=pltpu.CompilerParams(use_tc_tiling_on_sc=False),
)
def _sc_agg(e_hbm, hw_hbm, out_hbm, colbuf, rowbuf, vals, acc,
            hw_s, sem_i, sem_g, sem_s):
    cid = lax.axis_index("c")
    sid = lax.axis_index("s")
    wid = sid * NC + cid
    pltpu.sync_copy(hw_hbm.at[pl.ds(sid * RPT, RPT)],
                    acc.at[pl.ds(sid * RPT, RPT)])
    pltpu.sync_copy(hw_hbm.at[pl.ds(sid * RPT, RPT)],
                    hw_s.at[pl.ds(sid * RPT, RPT)])

    def real(m):
        return wid * GPT + m * GCH < EG

    def idx_copies(m):
        q = m % 3
        gbase = wid * GPT + m * GCH
        return (
            pltpu.make_async_copy(e_hbm.at[1, pl.ds(gbase, GCH)],
                                  colbuf.at[q], sem_i),
            pltpu.make_async_copy(e_hbm.at[0, pl.ds(gbase, GCH)],
                                  rowbuf.at[q], sem_i),
        )

    def fix_self_loops(m):
        q = m % 3
        for g in range(GCH):
            for jj in range(128 // 16):
                r = rowbuf[q, g, pl.ds(16 * jj, 16)]
                c = colbuf[q, g, pl.ds(16 * jj, 16)]
                rowbuf[q, g, pl.ds(16 * jj, 16)] = jnp.where(
                    r == c, TRASH, r)

    def gather_copies(m):
        q, b = m % 3, m % 2
        return [pltpu.make_async_copy(hw_s.at[colbuf.at[q, g]],
                                      vals.at[b, g], sem_g)
                for g in range(GCH)]

    def scatter_copies(m):
        q, b = m % 3, m % 2
        return [pltpu.make_async_copy(vals.at[b, g],
                                      acc.at[rowbuf.at[q, g]], sem_s)
                for g in range(GCH)]

    for c in idx_copies(0):
        c.start()
    plsc.subcore_barrier()
    for m in range(MACROS):
        if m >= 2:
            @pl.when(real(m - 2))
            def _(m=m):
                for c in scatter_copies(m - 2):
                    c.wait()

        @pl.when(real(m))
        def _(m=m):
            for c in idx_copies(m):
                c.wait()

        if m + 1 < MACROS:
            @pl.when(real(m + 1))
            def _(m=m):
                for c in idx_copies(m + 1):
                    c.start()

        @pl.when(real(m))
        def _(m=m):
            fix_self_loops(m)
            for c in gather_copies(m):
                c.start()

        if m >= 1:
            @pl.when(real(m - 1))
            def _(m=m):
                for c in gather_copies(m - 1):
                    c.wait()
                for c in scatter_copies(m - 1):
                    c.start(add=True)

    @pl.when(real(MACROS - 1))
    def _():
        for c in gather_copies(MACROS - 1):
            c.wait()
        for c in scatter_copies(MACROS - 1):
            c.start(add=True)

    @pl.when(real(MACROS - 2))
    def _():
        for c in scatter_copies(MACROS - 2):
            c.wait()

    @pl.when(real(MACROS - 1))
    def _():
        for c in scatter_copies(MACROS - 1):
            c.wait()
    plsc.subcore_barrier()
    pltpu.sync_copy(acc.at[pl.ds(sid * RPT, RPT)],
                    out_hbm.at[cid, pl.ds(sid * RPT, RPT)])


def _slot_pool(b4, h_p, width):
    seg = lax.broadcasted_iota(jnp.int32, (1, G), 1)
    s = jnp.zeros((G, width), jnp.float32)
    cnt = jnp.zeros((G, 1), jnp.float32)
    for j in range(4):
        Pj = (b4[:, j:j + 1] == seg).astype(jnp.float32)
        s = s + lax.dot_general(Pj, h_p[:, j * width:(j + 1) * width],
                                (((0,), (0,)), ((), ())),
                                preferred_element_type=jnp.float32)
        cnt = cnt + jnp.sum(Pj, axis=0)[:, None]
    return s / jnp.maximum(cnt, 1.0)


def _blockdiag(w, n_in, n_out):
    z = jnp.zeros((n_in, n_out), jnp.float32)
    rows = [jnp.concatenate([w if i == j else z for j in range(4)], axis=1)
            for i in range(4)]
    return jnp.concatenate(rows, axis=0)


def _head_body(x4_ref, w0_ref, b4_ref, hw_ref, pool_ref):
    x4 = x4_ref[...]
    hw_ref[...] = jnp.dot(x4, _blockdiag(w0_ref[...], F, DIM),
                          preferred_element_type=jnp.float32)
    pool_ref[...] = _slot_pool(b4_ref[...], x4, F)


_head = pl.pallas_call(
    _head_body,
    out_shape=(jax.ShapeDtypeStruct((RP, 4 * DIM), jnp.float32),
               jax.ShapeDtypeStruct((G, F), jnp.float32)),
)


def _bn_relu_packed(t, gamma, beta, mix):
    riota = lax.broadcasted_iota(jnp.int32, (RP, 1), 0)
    u = jnp.where(riota < N // 4, jnp.maximum(t, 0.0), 0.0)
    s1 = jnp.sum(u, axis=0, keepdims=True)
    s2 = jnp.sum(u * u, axis=0, keepdims=True)
    mu = jnp.dot(s1, mix, preferred_element_type=jnp.float32) / N
    m2 = jnp.dot(s2, mix, preferred_element_type=jnp.float32) / N
    var = m2 - mu * mu
    return gamma * (u - mu) / jnp.sqrt(var + 1e-5) + beta


def _tile4(v):
    return jnp.concatenate([v, v, v, v], axis=1)


def _mlp_body(parts_ref, hw_ref, b0_ref, g0_ref, be0_ref,
              w1_ref, b1_ref, g1_ref, be1_ref, wn_ref, b4_ref,
              hwn_ref, pool_ref):
    i = lax.broadcasted_iota(jnp.int32, (DIM, DIM), 0)
    j = lax.broadcasted_iota(jnp.int32, (DIM, DIM), 1)
    eye = (i == j).astype(jnp.float32)
    mix = jnp.concatenate([jnp.concatenate([eye] * 4, axis=1)] * 4, axis=0)
    agg = parts_ref[0] + parts_ref[1] - hw_ref[...]
    h = _bn_relu_packed(agg + _tile4(b0_ref[...]), _tile4(g0_ref[...]),
                        _tile4(be0_ref[...]), mix)
    t = jnp.dot(h, _blockdiag(w1_ref[...], DIM, DIM),
                preferred_element_type=jnp.float32) + _tile4(b1_ref[...])
    h2 = _bn_relu_packed(t, _tile4(g1_ref[...]), _tile4(be1_ref[...]), mix)
    hwn_ref[...] = jnp.dot(h2, _blockdiag(wn_ref[...], DIM, DIM),
                           preferred_element_type=jnp.float32)
    pool_ref[...] = _slot_pool(b4_ref[...], h2, DIM)


_mlp = pl.pallas_call(
    _mlp_body,
    out_shape=(jax.ShapeDtypeStruct((RP, 4 * DIM), jnp.float32),
               jax.ShapeDtypeStruct((G, DIM), jnp.float32)),
)


def _cls_body(px_ref, q0_ref, q1_ref, q2_ref, wc1_ref, bc1_ref,
              wc2_ref, bc2_ref, out_ref):
    z = jnp.concatenate(
        [px_ref[...], q0_ref[...], q1_ref[...], q2_ref[...]], axis=1)
    h = jnp.maximum(
        jnp.dot(z, wc1_ref[...], preferred_element_type=jnp.float32)
        + bc1_ref[...], 0.0)
    o = (jnp.dot(h, wc2_ref[...], preferred_element_type=jnp.float32)
         + bc2_ref[...])
    m = jnp.max(o, axis=1, keepdims=True)
    e = jnp.exp(o - m)
    out_ref[...] = (o - m) - jnp.log(jnp.sum(e, axis=1, keepdims=True))


_cls = pl.pallas_call(
    _cls_body,
    out_shape=jax.ShapeDtypeStruct((G, C), jnp.float32),
)


def kernel(x, edge_index, batch, params):
    e2d = edge_index.reshape(2, EG, 128)
    p = params

    x4 = jnp.concatenate(
        [x, jnp.zeros((NPAD - N, F), jnp.float32)]).reshape(RP, 4 * F)
    b4 = jnp.concatenate(
        [batch, jnp.full((NPAD - N,), G, jnp.int32)]).reshape(RP, 4)
    hw, poolx = _head(x4, p["W0_0"], b4)
    pools = [poolx]
    for k in range(L):
        parts = _sc_agg(e2d, hw.reshape(NPAD, DIM))
        wn = p[f"W{k + 1}_0"] if k < L - 1 else p["W1_0"]
        hw, pk = _mlp(
            parts.reshape(NC, RP, 4 * DIM), hw,
            p[f"b{k}_0"][None], p[f"g{k}_0"][None], p[f"be{k}_0"][None],
            p[f"W{k}_1"],
            p[f"b{k}_1"][None], p[f"g{k}_1"][None], p[f"be{k}_1"][None],
            wn, b4)
        pools.append(pk)
    return _cls(pools[0], pools[1], pools[2], pools[3],
                p["Wc1"], p["bc1"][None], p["Wc2"], p["bc2"][None])
